# baseline (device time: 165527 ns/iter reference)
import jax
import jax.numpy as jnp
from jax import lax
from jax.experimental import pallas as pl
from jax.experimental.pallas import tpu as pltpu


N_CHUNKS = 8
GRID = N_CHUNKS + 2


def kernel(dy, W):
    M, F = dy.shape
    D = W.shape[0]
    MH = M // 2
    DC = D // N_CHUNKS

    my_y0 = lax.axis_index("y")
    dy_half = lax.dynamic_slice_in_dim(dy, my_y0 * MH, MH, axis=0)

    def body(dy_ref, w_ref, out_ref,
             sx_ref, rx_ref, sy_ref, ry_ref,
             x_send, x_recv, y_send, y_recv,
             credit_x, credit_y):
        c = pl.program_id(0)
        my_x = lax.axis_index("x")
        my_y = lax.axis_index("y")
        x_nbr = (1 - my_x, my_y)
        y_nbr = (my_x, 1 - my_y)

        def x_rdma(slot):
            return pltpu.make_async_remote_copy(
                src_ref=sx_ref.at[slot], dst_ref=rx_ref.at[slot],
                send_sem=x_send.at[slot], recv_sem=x_recv.at[slot],
                device_id=x_nbr, device_id_type=pl.DeviceIdType.MESH)

        def y_rdma(slot):
            return pltpu.make_async_remote_copy(
                src_ref=sy_ref.at[slot], dst_ref=ry_ref.at[slot],
                send_sem=y_send.at[slot], recv_sem=y_recv.at[slot],
                device_id=y_nbr, device_id_type=pl.DeviceIdType.MESH)

        @pl.when(c == 0)
        def _barrier():
            bsem = pltpu.get_barrier_semaphore()
            pl.semaphore_signal(
                bsem, inc=1, device_id=x_nbr,
                device_id_type=pl.DeviceIdType.MESH)
            pl.semaphore_signal(
                bsem, inc=1, device_id=y_nbr,
                device_id_type=pl.DeviceIdType.MESH)
            pl.semaphore_wait(bsem, 2)

        @pl.when(c < N_CHUNKS)
        def _phase_a():
            slot = lax.rem(c, 2)
            p = lax.dot_general(
                dy_ref[...], w_ref[...],
                (((1,), (1,)), ((), ())),
                preferred_element_type=jnp.float32,
                precision=lax.Precision.DEFAULT)

            @pl.when(c >= 2)
            def _():
                x_rdma(slot).wait_send()
                pl.semaphore_wait(credit_x, 1)

            sx_ref[slot] = p
            x_rdma(slot).start()

        @pl.when((c >= 1) & (c <= N_CHUNKS))
        def _phase_b():
            slot = lax.rem(c - 1, 2)
            x_rdma(slot).wait_recv()
            r = sx_ref[slot] + rx_ref[slot]

            @pl.when(c >= 3)
            def _():
                y_rdma(slot).wait_send()
                pl.semaphore_wait(credit_y, 1)

            sy_ref[slot] = r
            y_rdma(slot).start()

            @pl.when(c <= 6)
            def _():
                pl.semaphore_signal(
                    credit_x, inc=1, device_id=x_nbr,
                    device_id_type=pl.DeviceIdType.MESH)

        @pl.when(c >= 2)
        def _phase_c():
            slot = lax.rem(c - 2, 2)
            y_rdma(slot).wait_recv()
            out_ref[pl.ds(my_y * MH, MH), :] = sy_ref[slot]
            out_ref[pl.ds((1 - my_y) * MH, MH), :] = ry_ref[slot]

            @pl.when(c <= 7)
            def _():
                pl.semaphore_signal(
                    credit_y, inc=1, device_id=y_nbr,
                    device_id_type=pl.DeviceIdType.MESH)

        @pl.when(c == GRID - 1)
        def _drain():
            for s in (0, 1):
                x_rdma(s).wait_send()
                y_rdma(s).wait_send()

    return pl.pallas_call(
        body,
        grid=(GRID,),
        out_shape=jax.ShapeDtypeStruct((M, D), jnp.float32),
        in_specs=[
            pl.BlockSpec(memory_space=pltpu.VMEM),
            pl.BlockSpec(
                (DC, F), lambda c: (jnp.minimum(c, N_CHUNKS - 1), 0)),
        ],
        out_specs=pl.BlockSpec(
            (M, DC), lambda c: (0, jnp.clip(c - 2, 0, N_CHUNKS - 1))),
        scratch_shapes=[
            pltpu.VMEM((2, MH, DC), jnp.float32),
            pltpu.VMEM((2, MH, DC), jnp.float32),
            pltpu.VMEM((2, MH, DC), jnp.float32),
            pltpu.VMEM((2, MH, DC), jnp.float32),
            pltpu.SemaphoreType.DMA((2,)),
            pltpu.SemaphoreType.DMA((2,)),
            pltpu.SemaphoreType.DMA((2,)),
            pltpu.SemaphoreType.DMA((2,)),
            pltpu.SemaphoreType.REGULAR,
            pltpu.SemaphoreType.REGULAR,
        ],
        compiler_params=pltpu.CompilerParams(
            collective_id=0,
            dimension_semantics=("arbitrary",),
            vmem_limit_bytes=64 * 1024 * 1024,
        ),
    )(dy_half, W)


# device time: 139859 ns/iter; 1.1835x vs baseline; 1.1835x over previous
import jax
import jax.numpy as jnp
from jax import lax
from jax.experimental import pallas as pl
from jax.experimental.pallas import tpu as pltpu


N_CHUNKS = 8
GRID = N_CHUNKS + 2


def kernel(dy, W):
    M, F = dy.shape
    D = W.shape[0]
    MH = M // 2
    DC = D // N_CHUNKS

    def body(y_ref, dy_ref, w_ref, out_ref,
             sx_ref, rx_ref, sy_ref, ry_ref,
             x_send, x_recv, y_send, y_recv,
             credit_x, credit_y, out_sem):
        c = pl.program_id(0)
        my_x = lax.axis_index("x")
        my_y = lax.axis_index("y")
        x_nbr = (1 - my_x, my_y)
        y_nbr = (my_x, 1 - my_y)

        def x_rdma(slot):
            return pltpu.make_async_remote_copy(
                src_ref=sx_ref.at[slot], dst_ref=rx_ref.at[slot],
                send_sem=x_send.at[slot], recv_sem=x_recv.at[slot],
                device_id=x_nbr, device_id_type=pl.DeviceIdType.MESH)

        def y_rdma(slot):
            return pltpu.make_async_remote_copy(
                src_ref=sy_ref.at[slot], dst_ref=ry_ref.at[slot],
                send_sem=y_send.at[slot], recv_sem=y_recv.at[slot],
                device_id=y_nbr, device_id_type=pl.DeviceIdType.MESH)

        @pl.when(c == 0)
        def _barrier():
            bsem = pltpu.get_barrier_semaphore()
            pl.semaphore_signal(
                bsem, inc=1, device_id=x_nbr,
                device_id_type=pl.DeviceIdType.MESH)
            pl.semaphore_signal(
                bsem, inc=1, device_id=y_nbr,
                device_id_type=pl.DeviceIdType.MESH)
            pl.semaphore_wait(bsem, 2)

        @pl.when(c < N_CHUNKS)
        def _phase_a():
            slot = lax.rem(c, 2)
            p = lax.dot_general(
                dy_ref[...], w_ref[...],
                (((1,), (1,)), ((), ())),
                preferred_element_type=jnp.float32,
                precision=lax.Precision.DEFAULT)

            @pl.when(c >= 2)
            def _():
                x_rdma(slot).wait_send()
                pl.semaphore_wait(credit_x, 1)

            sx_ref[slot] = p
            x_rdma(slot).start()

        @pl.when((c >= 1) & (c <= N_CHUNKS))
        def _phase_b():
            slot = lax.rem(c - 1, 2)
            x_rdma(slot).wait_recv()

            @pl.when(c >= 3)
            def _():
                y_rdma(slot).wait_send()
                pl.semaphore_wait(credit_y, 1)

            sy_ref[slot] = sx_ref[slot] + rx_ref[slot]
            y_rdma(slot).start()

            @pl.when(c <= 6)
            def _():
                pl.semaphore_signal(
                    credit_x, inc=1, device_id=x_nbr,
                    device_id_type=pl.DeviceIdType.MESH)

        @pl.when(c >= 2)
        def _phase_c():
            k = c - 2
            slot = lax.rem(k, 2)
            y_rdma(slot).wait_recv()
            cp_mine = pltpu.make_async_copy(
                sy_ref.at[slot],
                out_ref.at[pl.ds(my_y * MH, MH), pl.ds(k * DC, DC)],
                out_sem.at[0])
            cp_theirs = pltpu.make_async_copy(
                ry_ref.at[slot],
                out_ref.at[pl.ds((1 - my_y) * MH, MH), pl.ds(k * DC, DC)],
                out_sem.at[1])
            cp_mine.start()
            cp_theirs.start()
            cp_mine.wait()
            cp_theirs.wait()

            @pl.when(c <= 7)
            def _():
                pl.semaphore_signal(
                    credit_y, inc=1, device_id=y_nbr,
                    device_id_type=pl.DeviceIdType.MESH)

        @pl.when(c == GRID - 1)
        def _drain():
            for s in (0, 1):
                x_rdma(s).wait_send()
                y_rdma(s).wait_send()

    my_y0 = lax.axis_index("y").astype(jnp.int32).reshape((1,))

    grid_spec = pltpu.PrefetchScalarGridSpec(
        num_scalar_prefetch=1,
        grid=(GRID,),
        in_specs=[
            pl.BlockSpec((MH, F), lambda c, y: (y[0], 0)),
            pl.BlockSpec(
                (DC, F), lambda c, y: (jnp.minimum(c, N_CHUNKS - 1), 0)),
        ],
        out_specs=pl.BlockSpec(memory_space=pl.ANY),
        scratch_shapes=[
            pltpu.VMEM((2, MH, DC), jnp.float32),
            pltpu.VMEM((2, MH, DC), jnp.float32),
            pltpu.VMEM((2, MH, DC), jnp.float32),
            pltpu.VMEM((2, MH, DC), jnp.float32),
            pltpu.SemaphoreType.DMA((2,)),
            pltpu.SemaphoreType.DMA((2,)),
            pltpu.SemaphoreType.DMA((2,)),
            pltpu.SemaphoreType.DMA((2,)),
            pltpu.SemaphoreType.REGULAR,
            pltpu.SemaphoreType.REGULAR,
            pltpu.SemaphoreType.DMA((2,)),
        ],
    )

    return pl.pallas_call(
        body,
        grid_spec=grid_spec,
        out_shape=jax.ShapeDtypeStruct((M, D), jnp.float32),
        compiler_params=pltpu.CompilerParams(
            collective_id=0,
            dimension_semantics=("arbitrary",),
            vmem_limit_bytes=64 * 1024 * 1024,
        ),
    )(my_y0, dy, W)


# device time: 139659 ns/iter; 1.1852x vs baseline; 1.0014x over previous
import jax
import jax.numpy as jnp
from jax import lax
from jax.experimental import pallas as pl
from jax.experimental.pallas import tpu as pltpu


N_CHUNKS = 8
GRID = N_CHUNKS + 2


def kernel(dy, W):
    M, F = dy.shape
    D = W.shape[0]
    MH = M // 2
    DC = D // N_CHUNKS

    def body(y_ref, dy_ref, w_ref, out_ref,
             sx_ref, rx_ref, sy_ref, ry_ref,
             x_send, x_recv, y_send, y_recv,
             credit_x, credit_y, out_sem):
        c = pl.program_id(0)
        my_x = lax.axis_index("x")
        my_y = lax.axis_index("y")
        x_nbr = (1 - my_x, my_y)
        y_nbr = (my_x, 1 - my_y)

        def x_rdma(slot):
            return pltpu.make_async_remote_copy(
                src_ref=sx_ref.at[slot], dst_ref=rx_ref.at[slot],
                send_sem=x_send.at[slot], recv_sem=x_recv.at[slot],
                device_id=x_nbr, device_id_type=pl.DeviceIdType.MESH)

        def y_rdma(slot):
            return pltpu.make_async_remote_copy(
                src_ref=sy_ref.at[slot], dst_ref=ry_ref.at[slot],
                send_sem=y_send.at[slot], recv_sem=y_recv.at[slot],
                device_id=y_nbr, device_id_type=pl.DeviceIdType.MESH)

        def out_cp(k):
            s3 = lax.rem(k, 3)
            s2 = lax.rem(k, 2)
            col = pl.ds(k * DC, DC)
            mine = pltpu.make_async_copy(
                sy_ref.at[s3],
                out_ref.at[pl.ds(my_y * MH, MH), col],
                out_sem.at[s2, 0])
            theirs = pltpu.make_async_copy(
                ry_ref.at[s3],
                out_ref.at[pl.ds((1 - my_y) * MH, MH), col],
                out_sem.at[s2, 1])
            return mine, theirs

        @pl.when(c == 0)
        def _barrier():
            bsem = pltpu.get_barrier_semaphore()
            pl.semaphore_signal(
                bsem, inc=1, device_id=x_nbr,
                device_id_type=pl.DeviceIdType.MESH)
            pl.semaphore_signal(
                bsem, inc=1, device_id=y_nbr,
                device_id_type=pl.DeviceIdType.MESH)
            pl.semaphore_wait(bsem, 2)

        @pl.when(c < N_CHUNKS)
        def _phase_a():
            slot = lax.rem(c, 2)
            p = lax.dot_general(
                dy_ref[...], w_ref[...],
                (((1,), (1,)), ((), ())),
                preferred_element_type=jnp.float32,
                precision=lax.Precision.DEFAULT)

            @pl.when(c >= 2)
            def _():
                x_rdma(slot).wait_send()
                pl.semaphore_wait(credit_x, 1)

            sx_ref[slot] = p
            x_rdma(slot).start()

        @pl.when((c >= 1) & (c <= N_CHUNKS))
        def _phase_b():
            xslot = lax.rem(c - 1, 2)
            yslot = lax.rem(c - 1, 3)
            x_rdma(xslot).wait_recv()

            @pl.when(c >= 4)
            def _():
                y_rdma(yslot).wait_send()
                pl.semaphore_wait(credit_y, 1)

            sy_ref[yslot] = sx_ref[xslot] + rx_ref[xslot]
            y_rdma(yslot).start()

            @pl.when(c <= 6)
            def _():
                pl.semaphore_signal(
                    credit_x, inc=1, device_id=x_nbr,
                    device_id_type=pl.DeviceIdType.MESH)

        @pl.when(c >= 2)
        def _phase_c():
            k = c - 2

            @pl.when(c >= 3)
            def _():
                pm, pt = out_cp(jnp.maximum(k - 1, 0))
                pm.wait()
                pt.wait()
                @pl.when(c <= 7)
                def _():
                    pl.semaphore_signal(
                        credit_y, inc=1, device_id=y_nbr,
                        device_id_type=pl.DeviceIdType.MESH)

            y_rdma(lax.rem(k, 3)).wait_recv()
            mine, theirs = out_cp(k)
            mine.start()
            theirs.start()

            @pl.when(c == GRID - 1)
            def _():
                mine2, theirs2 = out_cp(k)
                mine2.wait()
                theirs2.wait()

        @pl.when(c == GRID - 1)
        def _drain():
            for s in (0, 1):
                x_rdma(s).wait_send()
            for s in (0, 1, 2):
                y_rdma(s).wait_send()

    my_y0 = lax.axis_index("y").astype(jnp.int32).reshape((1,))

    grid_spec = pltpu.PrefetchScalarGridSpec(
        num_scalar_prefetch=1,
        grid=(GRID,),
        in_specs=[
            pl.BlockSpec((MH, F), lambda c, y: (y[0], 0)),
            pl.BlockSpec(
                (DC, F), lambda c, y: (jnp.minimum(c, N_CHUNKS - 1), 0)),
        ],
        out_specs=pl.BlockSpec(memory_space=pl.ANY),
        scratch_shapes=[
            pltpu.VMEM((2, MH, DC), jnp.float32),
            pltpu.VMEM((2, MH, DC), jnp.float32),
            pltpu.VMEM((3, MH, DC), jnp.float32),
            pltpu.VMEM((3, MH, DC), jnp.float32),
            pltpu.SemaphoreType.DMA((2,)),
            pltpu.SemaphoreType.DMA((2,)),
            pltpu.SemaphoreType.DMA((3,)),
            pltpu.SemaphoreType.DMA((3,)),
            pltpu.SemaphoreType.REGULAR,
            pltpu.SemaphoreType.REGULAR,
            pltpu.SemaphoreType.DMA((2, 2)),
        ],
    )

    return pl.pallas_call(
        body,
        grid_spec=grid_spec,
        out_shape=jax.ShapeDtypeStruct((M, D), jnp.float32),
        compiler_params=pltpu.CompilerParams(
            collective_id=0,
            dimension_semantics=("arbitrary",),
            vmem_limit_bytes=64 * 1024 * 1024,
        ),
    )(my_y0, dy, W)
